# baseline (speedup 1.0000x reference)
import functools

import jax
import jax.numpy as jnp
from jax import lax
from jax.experimental import pallas as pl
from jax.experimental.pallas import tpu as pltpu
from jax.experimental.pallas import tpu_sc as plsc

S = 100000
DS = 32
N = 524288
DB = 26
DE = 26
H = 32
CP = 32


_S_BLK = 2000


def _mlp_body(sf, wb1, bb1, wb2, bb2, we1, be1, we2, be2, tb_out, te_out):
    x = sf[...]
    hb = jax.nn.silu(jnp.dot(x, wb1[...], preferred_element_type=jnp.float32)
                     + bb1[...])
    tb_out[...] = (jnp.dot(hb, wb2[...], preferred_element_type=jnp.float32)
                   + bb2[...])
    he = jax.nn.silu(jnp.dot(x, we1[...], preferred_element_type=jnp.float32)
                     + be1[...])
    te_out[...] = (jnp.dot(he, we2[...], preferred_element_type=jnp.float32)
                   + be2[...])


def _mlp_tables(sf, Wb1, bb1, Wb2, bb2, We1, be1, We2, be2):
    Wb2p = jnp.pad(Wb2, ((0, 0), (0, CP - DB)))
    bb2p = jnp.pad(bb2, (0, CP - DB)).reshape(1, CP)
    We2p = jnp.pad(We2, ((0, 0), (0, CP - DE)))
    be2p = jnp.pad(be2, (0, CP - DE)).reshape(1, CP)
    bb1r = bb1.reshape(1, H)
    be1r = be1.reshape(1, H)

    grid = S // _S_BLK
    full = lambda i: (0, 0)
    return pl.pallas_call(
        _mlp_body,
        grid=(grid,),
        in_specs=[
            pl.BlockSpec((_S_BLK, DS), lambda i: (i, 0)),
            pl.BlockSpec((DS, H), full),
            pl.BlockSpec((1, H), full),
            pl.BlockSpec((H, CP), full),
            pl.BlockSpec((1, CP), full),
            pl.BlockSpec((DS, H), full),
            pl.BlockSpec((1, H), full),
            pl.BlockSpec((H, CP), full),
            pl.BlockSpec((1, CP), full),
        ],
        out_specs=[
            pl.BlockSpec((_S_BLK, CP), lambda i: (i, 0)),
            pl.BlockSpec((_S_BLK, CP), lambda i: (i, 0)),
        ],
        out_shape=[
            jax.ShapeDtypeStruct((S, CP), jnp.float32),
            jax.ShapeDtypeStruct((S, CP), jnp.float32),
        ],
    )(sf, Wb1, bb1r, Wb2p, bb2p, We1, be1r, We2p, be2p)


_NC = 2
_NS = 16
_NW = _NC * _NS
_ROWS_W = N // _NW
_CHUNK = 512
_NCHUNK = _ROWS_W // _CHUNK
_NGRP = _CHUNK // 16


def _sc_fused(loc_ind, tb, te, bf, ef, csm, alert):
    mesh = plsc.VectorSubcoreMesh(core_axis_name="c", subcore_axis_name="s")

    @functools.partial(
        pl.kernel,
        mesh=mesh,
        out_type=(
            jax.ShapeDtypeStruct((N,), jnp.float32),
            jax.ShapeDtypeStruct((N,), jnp.float32),
            jax.ShapeDtypeStruct((N,), jnp.float32),
        ),
        scratch_types=[
            pltpu.VMEM((_ROWS_W,), jnp.int32),
            pltpu.VMEM((_CHUNK, CP), jnp.float32),
            pltpu.VMEM((_CHUNK, CP), jnp.float32),
            pltpu.VMEM((_CHUNK, DB), jnp.float32),
            pltpu.VMEM((_CHUNK, DE), jnp.float32),
            pltpu.VMEM((_CHUNK,), jnp.float32),
            pltpu.VMEM((_CHUNK,), jnp.float32),
            pltpu.VMEM((_CHUNK,), jnp.float32),
            pltpu.VMEM((_CHUNK,), jnp.float32),
            pltpu.VMEM((_CHUNK,), jnp.float32),
            pltpu.SemaphoreType.DMA,
            pltpu.SemaphoreType.DMA,
        ],
        compiler_params=pltpu.CompilerParams(use_tc_tiling_on_sc=False,
                                             needs_layout_passes=False),
    )
    def k(idx_hbm, tb_hbm, te_hbm, bf_hbm, ef_hbm, csm_hbm, al_hbm,
          eff_hbm, base_hbm, outc_hbm,
          idx_v, rb_v, re_v, bf_v, ef_v, csm_v, al_v, eff_v, base_v, outc_v,
          semb, seme):
        wid = lax.axis_index("s") * _NC + lax.axis_index("c")
        wbase = wid * _ROWS_W
        pltpu.sync_copy(idx_hbm.at[pl.ds(wbase, _ROWS_W)], idx_v)
        lane = lax.iota(jnp.int32, 16)

        def chunk(t, carry):
            off = t * _CHUNK
            base = wbase + off
            ids = idx_v.at[pl.ds(off, _CHUNK)]
            cb = pltpu.async_copy(tb_hbm.at[ids], rb_v, semb)
            ce = pltpu.async_copy(te_hbm.at[ids], re_v, seme)
            pltpu.sync_copy(bf_hbm.at[pl.ds(base, _CHUNK)], bf_v)
            pltpu.sync_copy(ef_hbm.at[pl.ds(base, _CHUNK)], ef_v)
            pltpu.sync_copy(csm_hbm.at[pl.ds(base, _CHUNK)], csm_v)
            pltpu.sync_copy(al_hbm.at[pl.ds(base, _CHUNK)], al_v)
            cb.wait()
            ce.wait()

            def group(g, c2):
                r = g * 16 + lane
                acc_b = jnp.zeros((16,), jnp.float32)
                acc_e = jnp.zeros((16,), jnp.float32)
                for j in range(DB):
                    js = jnp.full((16,), j, jnp.int32)
                    acc_b += (plsc.load_gather(rb_v, [r, js])
                              * plsc.load_gather(bf_v, [r, js]))
                    acc_e += (plsc.load_gather(re_v, [r, js])
                              * plsc.load_gather(ef_v, [r, js]))
                baseline = jnp.minimum(jnp.exp(acc_b), 1e6)
                eff = 1.0 / (1.0 + jnp.exp(4.0 - acc_e))
                eff = jnp.clip(eff, 1e-6, 1.0 - 1e-6)
                sl = pl.ds(g * 16, 16)
                csm16 = csm_v[sl]
                al16 = al_v[sl]
                eff_v[sl] = eff
                base_v[sl] = baseline
                outc_v[sl] = csm16 * baseline * (1.0 - al16 * eff)
                return c2

            lax.fori_loop(0, _NGRP, group, 0)
            pltpu.sync_copy(eff_v, eff_hbm.at[pl.ds(base, _CHUNK)])
            pltpu.sync_copy(base_v, base_hbm.at[pl.ds(base, _CHUNK)])
            pltpu.sync_copy(outc_v, outc_hbm.at[pl.ds(base, _CHUNK)])
            return carry

        lax.fori_loop(0, _NCHUNK, chunk, 0)

    return k(loc_ind, tb, te, bf, ef, csm, alert)


def kernel(hosps, loc_ind, county_summer_mean, alert, baseline_features,
           eff_features, index, spatial_features,
           Wb1, bb1, Wb2, bb2, We1, be1, We2, be2):
    tb, te = _mlp_tables(spatial_features, Wb1, bb1, Wb2, bb2,
                         We1, be1, We2, be2)
    eff, base, outc = _sc_fused(loc_ind, tb, te, baseline_features,
                                eff_features, county_summer_mean, alert)
    return jnp.stack([eff, base, outc], axis=1)


# speedup vs baseline: 1.2571x; 1.2571x over previous
import functools

import jax
import jax.numpy as jnp
from jax import lax
from jax.experimental import pallas as pl
from jax.experimental.pallas import tpu as pltpu
from jax.experimental.pallas import tpu_sc as plsc

S = 100000
DS = 32
N = 524288
DB = 26
DE = 26
H = 32
CP = 128
EOFF = 64


_S_BLK = 2000


def _mlp_body(sf, wb1, bb1, wb2, bb2, we1, be1, we2, be2, tbl_out):
    x = sf[...]
    hb = jax.nn.silu(jnp.dot(x, wb1[...], preferred_element_type=jnp.float32)
                     + bb1[...])
    b = jnp.dot(hb, wb2[...], preferred_element_type=jnp.float32) + bb2[...]
    he = jax.nn.silu(jnp.dot(x, we1[...], preferred_element_type=jnp.float32)
                     + be1[...])
    e = jnp.dot(he, we2[...], preferred_element_type=jnp.float32) + be2[...]
    tbl_out[...] = jnp.concatenate([b, e], axis=1)


def _mlp_table(sf, Wb1, bb1, Wb2, bb2, We1, be1, We2, be2):
    Wb2p = jnp.pad(Wb2, ((0, 0), (0, EOFF - DB)))
    bb2p = jnp.pad(bb2, (0, EOFF - DB)).reshape(1, EOFF)
    We2p = jnp.pad(We2, ((0, 0), (0, EOFF - DE)))
    be2p = jnp.pad(be2, (0, EOFF - DE)).reshape(1, EOFF)
    bb1r = bb1.reshape(1, H)
    be1r = be1.reshape(1, H)

    grid = S // _S_BLK
    full = lambda i: (0, 0)
    return pl.pallas_call(
        _mlp_body,
        grid=(grid,),
        in_specs=[
            pl.BlockSpec((_S_BLK, DS), lambda i: (i, 0)),
            pl.BlockSpec((DS, H), full),
            pl.BlockSpec((1, H), full),
            pl.BlockSpec((H, EOFF), full),
            pl.BlockSpec((1, EOFF), full),
            pl.BlockSpec((DS, H), full),
            pl.BlockSpec((1, H), full),
            pl.BlockSpec((H, EOFF), full),
            pl.BlockSpec((1, EOFF), full),
        ],
        out_specs=pl.BlockSpec((_S_BLK, CP), lambda i: (i, 0)),
        out_shape=jax.ShapeDtypeStruct((S, CP), jnp.float32),
    )(sf, Wb1, bb1r, Wb2p, bb2p, We1, be1r, We2p, be2p)


_NC = 2
_NS = 16
_NW = _NC * _NS
_ROWS_W = N // _NW
_CHUNK = 256
_NCHUNK = _ROWS_W // _CHUNK
_NGRP = _CHUNK // 16


def _sc_fused(loc_ind, tbl, bf, ef, csm, alert):
    mesh = plsc.VectorSubcoreMesh(core_axis_name="c", subcore_axis_name="s")

    @functools.partial(
        pl.kernel,
        mesh=mesh,
        out_type=(
            jax.ShapeDtypeStruct((N,), jnp.float32),
            jax.ShapeDtypeStruct((N,), jnp.float32),
            jax.ShapeDtypeStruct((N,), jnp.float32),
        ),
        scratch_types=[
            pltpu.VMEM((_CHUNK,), jnp.int32),
            pltpu.VMEM((_CHUNK, CP), jnp.float32),
            pltpu.VMEM((_CHUNK, DB), jnp.float32),
            pltpu.VMEM((_CHUNK, DE), jnp.float32),
            pltpu.VMEM((_CHUNK,), jnp.float32),
            pltpu.VMEM((_CHUNK,), jnp.float32),
            pltpu.VMEM((_CHUNK,), jnp.float32),
            pltpu.VMEM((_CHUNK,), jnp.float32),
            pltpu.VMEM((_CHUNK,), jnp.float32),
            pltpu.SemaphoreType.DMA,
        ],
        compiler_params=pltpu.CompilerParams(needs_layout_passes=False),
    )
    def k(idx_hbm, tbl_hbm, bf_hbm, ef_hbm, csm_hbm, al_hbm,
          eff_hbm, base_hbm, outc_hbm,
          idx_v, rows_v, bf_v, ef_v, csm_v, al_v, eff_v, base_v, outc_v,
          sem):
        wid = lax.axis_index("s") * _NC + lax.axis_index("c")
        wbase = wid * _ROWS_W
        lane = lax.iota(jnp.int32, 16)

        def chunk(t, carry):
            base = wbase + t * _CHUNK
            pltpu.sync_copy(idx_hbm.at[pl.ds(base, _CHUNK)], idx_v)
            cg = pltpu.async_copy(tbl_hbm.at[idx_v], rows_v, sem)
            pltpu.sync_copy(bf_hbm.at[pl.ds(base, _CHUNK)], bf_v)
            pltpu.sync_copy(ef_hbm.at[pl.ds(base, _CHUNK)], ef_v)
            pltpu.sync_copy(csm_hbm.at[pl.ds(base, _CHUNK)], csm_v)
            pltpu.sync_copy(al_hbm.at[pl.ds(base, _CHUNK)], al_v)
            cg.wait()

            @plsc.parallel_loop(0, _NGRP, 1)
            def group(g):
                r = g * 16 + lane
                acc_b = jnp.zeros((16,), jnp.float32)
                acc_e = jnp.zeros((16,), jnp.float32)
                for j in range(DB):
                    js = jnp.full((16,), j, jnp.int32)
                    jse = jnp.full((16,), EOFF + j, jnp.int32)
                    acc_b += (plsc.load_gather(rows_v, [r, js])
                              * plsc.load_gather(bf_v, [r, js]))
                    acc_e += (plsc.load_gather(rows_v, [r, jse])
                              * plsc.load_gather(ef_v, [r, js]))
                baseline = jnp.minimum(jnp.exp(acc_b), 1e6)
                eff = 1.0 / (1.0 + jnp.exp(4.0 - acc_e))
                eff = jnp.clip(eff, 1e-6, 1.0 - 1e-6)
                sl = pl.ds(g * 16, 16)
                csm16 = csm_v[sl]
                al16 = al_v[sl]
                eff_v[sl] = eff
                base_v[sl] = baseline
                outc_v[sl] = csm16 * baseline * (1.0 - al16 * eff)

            pltpu.sync_copy(eff_v, eff_hbm.at[pl.ds(base, _CHUNK)])
            pltpu.sync_copy(base_v, base_hbm.at[pl.ds(base, _CHUNK)])
            pltpu.sync_copy(outc_v, outc_hbm.at[pl.ds(base, _CHUNK)])
            return carry

        lax.fori_loop(0, _NCHUNK, chunk, 0)

    return k(loc_ind, tbl, bf, ef, csm, alert)


def kernel(hosps, loc_ind, county_summer_mean, alert, baseline_features,
           eff_features, index, spatial_features,
           Wb1, bb1, Wb2, bb2, We1, be1, We2, be2):
    tbl = _mlp_table(spatial_features, Wb1, bb1, Wb2, bb2,
                     We1, be1, We2, be2)
    eff, base, outc = _sc_fused(loc_ind, tbl, baseline_features,
                                eff_features, county_summer_mean, alert)
    return jnp.stack([eff, base, outc], axis=1)


# speedup vs baseline: 1.9907x; 1.5835x over previous
import functools

import jax
import jax.numpy as jnp
from jax import lax
from jax.experimental import pallas as pl
from jax.experimental.pallas import tpu as pltpu
from jax.experimental.pallas import tpu_sc as plsc

S = 100000
DS = 32
N = 524288
DB = 26
DE = 26
H = 32
CP = 128
EOFF = 64


_S_BLK = 2000


def _mlp_body(sf, wb1, bb1, wb2, bb2, we1, be1, we2, be2, tbl_out):
    x = sf[...]
    hb = jax.nn.silu(jnp.dot(x, wb1[...], preferred_element_type=jnp.float32)
                     + bb1[...])
    b = jnp.dot(hb, wb2[...], preferred_element_type=jnp.float32) + bb2[...]
    he = jax.nn.silu(jnp.dot(x, we1[...], preferred_element_type=jnp.float32)
                     + be1[...])
    e = jnp.dot(he, we2[...], preferred_element_type=jnp.float32) + be2[...]
    tbl_out[...] = jnp.concatenate([b, e], axis=1)


def _mlp_table(sf, Wb1, bb1, Wb2, bb2, We1, be1, We2, be2):
    Wb2p = jnp.pad(Wb2, ((0, 0), (0, EOFF - DB)))
    bb2p = jnp.pad(bb2, (0, EOFF - DB)).reshape(1, EOFF)
    We2p = jnp.pad(We2, ((0, 0), (0, EOFF - DE)))
    be2p = jnp.pad(be2, (0, EOFF - DE)).reshape(1, EOFF)
    bb1r = bb1.reshape(1, H)
    be1r = be1.reshape(1, H)

    grid = S // _S_BLK
    full = lambda i: (0, 0)
    return pl.pallas_call(
        _mlp_body,
        grid=(grid,),
        in_specs=[
            pl.BlockSpec((_S_BLK, DS), lambda i: (i, 0)),
            pl.BlockSpec((DS, H), full),
            pl.BlockSpec((1, H), full),
            pl.BlockSpec((H, EOFF), full),
            pl.BlockSpec((1, EOFF), full),
            pl.BlockSpec((DS, H), full),
            pl.BlockSpec((1, H), full),
            pl.BlockSpec((H, EOFF), full),
            pl.BlockSpec((1, EOFF), full),
        ],
        out_specs=pl.BlockSpec((_S_BLK, CP), lambda i: (i, 0)),
        out_shape=jax.ShapeDtypeStruct((S, CP), jnp.float32),
    )(sf, Wb1, bb1r, Wb2p, bb2p, We1, be1r, We2p, be2p)


_NC = 2
_NS = 16
_NW = _NC * _NS
_ROWS_W = N // _NW
_CHUNK = 256
_NCHUNK = _ROWS_W // _CHUNK
_NGRP = _CHUNK // 16


def _sc_fused(loc_ind, tbl, bf, ef, csm, alert):
    mesh = plsc.VectorSubcoreMesh(core_axis_name="c", subcore_axis_name="s")

    @functools.partial(
        pl.kernel,
        mesh=mesh,
        out_type=(
            jax.ShapeDtypeStruct((N,), jnp.float32),
            jax.ShapeDtypeStruct((N,), jnp.float32),
            jax.ShapeDtypeStruct((N,), jnp.float32),
        ),
        scratch_types=[
            pltpu.VMEM((_CHUNK,), jnp.int32),
            pltpu.VMEM((_CHUNK, CP), jnp.float32),
            pltpu.VMEM((_CHUNK, DB), jnp.float32),
            pltpu.VMEM((_CHUNK, DE), jnp.float32),
            pltpu.VMEM((_CHUNK,), jnp.float32),
            pltpu.VMEM((_CHUNK,), jnp.float32),
            pltpu.VMEM((_CHUNK,), jnp.float32),
            pltpu.VMEM((_CHUNK,), jnp.float32),
            pltpu.VMEM((_CHUNK,), jnp.float32),
            pltpu.SemaphoreType.DMA,
        ],
        compiler_params=pltpu.CompilerParams(needs_layout_passes=False),
    )
    def k(idx_hbm, tbl_hbm, bf_hbm, ef_hbm, csm_hbm, al_hbm,
          eff_hbm, base_hbm, outc_hbm,
          idx_v, rows_v, bf_v, ef_v, csm_v, al_v, eff_v, base_v, outc_v,
          sem):
        wid = lax.axis_index("s") * _NC + lax.axis_index("c")
        wbase = wid * _ROWS_W
        lane = lax.iota(jnp.int32, 16)

        def chunk(t, carry):
            base = wbase + t * _CHUNK
            pltpu.sync_copy(idx_hbm.at[pl.ds(base, _CHUNK)], idx_v)
            cg = pltpu.async_copy(tbl_hbm.at[idx_v], rows_v, sem)
            pltpu.sync_copy(bf_hbm.at[pl.ds(base, _CHUNK)], bf_v)
            pltpu.sync_copy(ef_hbm.at[pl.ds(base, _CHUNK)], ef_v)
            pltpu.sync_copy(csm_hbm.at[pl.ds(base, _CHUNK)], csm_v)
            pltpu.sync_copy(al_hbm.at[pl.ds(base, _CHUNK)], al_v)
            cg.wait()

            @plsc.parallel_loop(0, _NGRP, 1)
            def group(g):
                r = g * 16 + lane
                acc_b0 = jnp.zeros((16,), jnp.float32)
                acc_b1 = jnp.zeros((16,), jnp.float32)
                acc_e0 = jnp.zeros((16,), jnp.float32)
                acc_e1 = jnp.zeros((16,), jnp.float32)
                for d in range(DB):
                    js = lax.rem(lane + d, jnp.full((16,), DB, jnp.int32))
                    jse = js + EOFF
                    pb = (plsc.load_gather(rows_v, [r, js])
                          * plsc.load_gather(bf_v, [r, js]))
                    pe = (plsc.load_gather(rows_v, [r, jse])
                          * plsc.load_gather(ef_v, [r, js]))
                    if d % 2 == 0:
                        acc_b0 += pb
                        acc_e0 += pe
                    else:
                        acc_b1 += pb
                        acc_e1 += pe
                acc_b = acc_b0 + acc_b1
                acc_e = acc_e0 + acc_e1
                baseline = jnp.minimum(jnp.exp(acc_b), 1e6)
                eff = 1.0 / (1.0 + jnp.exp(4.0 - acc_e))
                eff = jnp.clip(eff, 1e-6, 1.0 - 1e-6)
                sl = pl.ds(g * 16, 16)
                csm16 = csm_v[sl]
                al16 = al_v[sl]
                eff_v[sl] = eff
                base_v[sl] = baseline
                outc_v[sl] = csm16 * baseline * (1.0 - al16 * eff)

            pltpu.sync_copy(eff_v, eff_hbm.at[pl.ds(base, _CHUNK)])
            pltpu.sync_copy(base_v, base_hbm.at[pl.ds(base, _CHUNK)])
            pltpu.sync_copy(outc_v, outc_hbm.at[pl.ds(base, _CHUNK)])
            return carry

        lax.fori_loop(0, _NCHUNK, chunk, 0)

    return k(loc_ind, tbl, bf, ef, csm, alert)


def kernel(hosps, loc_ind, county_summer_mean, alert, baseline_features,
           eff_features, index, spatial_features,
           Wb1, bb1, Wb2, bb2, We1, be1, We2, be2):
    tbl = _mlp_table(spatial_features, Wb1, bb1, Wb2, bb2,
                     We1, be1, We2, be2)
    eff, base, outc = _sc_fused(loc_ind, tbl, baseline_features,
                                eff_features, county_summer_mean, alert)
    return jnp.stack([eff, base, outc], axis=1)


# speedup vs baseline: 3.8052x; 1.9116x over previous
import functools

import jax
import jax.numpy as jnp
from jax import lax
from jax.experimental import pallas as pl
from jax.experimental.pallas import tpu as pltpu
from jax.experimental.pallas import tpu_sc as plsc

S = 100000
DS = 32
N = 524288
DB = 26
DE = 26
H = 32
CP = 128
EOFF = 64


_S_BLK = 2048


def _mlp_body(sft, wb1, bb1, wb2, bb2, we1, be1, we2, be2, tbl_out):
    xt = sft[...]
    dims = (((0,), (0,)), ((), ()))
    hb = jax.nn.silu(
        lax.dot_general(xt, wb1[...], dims,
                        preferred_element_type=jnp.float32) + bb1[...])
    b = jnp.dot(hb, wb2[...], preferred_element_type=jnp.float32) + bb2[...]
    he = jax.nn.silu(
        lax.dot_general(xt, we1[...], dims,
                        preferred_element_type=jnp.float32) + be1[...])
    e = jnp.dot(he, we2[...], preferred_element_type=jnp.float32) + be2[...]
    tbl_out[...] = jnp.concatenate([b, e], axis=1)


def _mlp_table(sft, Wb1, bb1, Wb2, bb2, We1, be1, We2, be2):
    Wb2p = jnp.pad(Wb2, ((0, 0), (0, EOFF - DB)))
    bb2p = jnp.pad(bb2, (0, EOFF - DB)).reshape(1, EOFF)
    We2p = jnp.pad(We2, ((0, 0), (0, EOFF - DE)))
    be2p = jnp.pad(be2, (0, EOFF - DE)).reshape(1, EOFF)
    bb1r = bb1.reshape(1, H)
    be1r = be1.reshape(1, H)

    grid = (S + _S_BLK - 1) // _S_BLK
    full = lambda i: (0, 0)
    return pl.pallas_call(
        _mlp_body,
        grid=(grid,),
        in_specs=[
            pl.BlockSpec((DS, _S_BLK), lambda i: (0, i)),
            pl.BlockSpec((DS, H), full),
            pl.BlockSpec((1, H), full),
            pl.BlockSpec((H, EOFF), full),
            pl.BlockSpec((1, EOFF), full),
            pl.BlockSpec((DS, H), full),
            pl.BlockSpec((1, H), full),
            pl.BlockSpec((H, EOFF), full),
            pl.BlockSpec((1, EOFF), full),
        ],
        out_specs=pl.BlockSpec((_S_BLK, CP), lambda i: (i, 0)),
        out_shape=jax.ShapeDtypeStruct((S, CP), jnp.float32),
    )(sft, Wb1, bb1r, Wb2p, bb2p, We1, be1r, We2p, be2p)


_NC = 2
_NS = 16
_NW = _NC * _NS
_ROWS_W = N // _NW
_CHUNK = 256
_NCHUNK = _ROWS_W // _CHUNK
_NGRP = _CHUNK // 16


def _sc_fused(loc_ind, tbl, bft, eft, csm, alert):
    mesh = plsc.VectorSubcoreMesh(core_axis_name="c", subcore_axis_name="s")

    @functools.partial(
        pl.kernel,
        mesh=mesh,
        out_type=(
            jax.ShapeDtypeStruct((N,), jnp.float32),
            jax.ShapeDtypeStruct((N,), jnp.float32),
            jax.ShapeDtypeStruct((N,), jnp.float32),
        ),
        scratch_types=[
            pltpu.VMEM((_CHUNK,), jnp.int32),
            pltpu.VMEM((_CHUNK, CP), jnp.float32),
            pltpu.VMEM((DB, _CHUNK), jnp.float32),
            pltpu.VMEM((DE, _CHUNK), jnp.float32),
            pltpu.VMEM((_CHUNK,), jnp.float32),
            pltpu.VMEM((_CHUNK,), jnp.float32),
            pltpu.VMEM((_CHUNK,), jnp.float32),
            pltpu.VMEM((_CHUNK,), jnp.float32),
            pltpu.VMEM((_CHUNK,), jnp.float32),
            pltpu.SemaphoreType.DMA,
        ],
        compiler_params=pltpu.CompilerParams(needs_layout_passes=False),
    )
    def k(idx_hbm, tbl_hbm, bft_hbm, eft_hbm, csm_hbm, al_hbm,
          eff_hbm, base_hbm, outc_hbm,
          idx_v, rows_v, bft_v, eft_v, csm_v, al_v, eff_v, base_v, outc_v,
          sem):
        wid = lax.axis_index("s") * _NC + lax.axis_index("c")
        wbase = wid * _ROWS_W
        lane = lax.iota(jnp.int32, 16)

        def chunk(t, carry):
            base = wbase + t * _CHUNK
            pltpu.sync_copy(idx_hbm.at[pl.ds(base, _CHUNK)], idx_v)
            cg = pltpu.async_copy(tbl_hbm.at[idx_v], rows_v, sem)
            pltpu.sync_copy(bft_hbm.at[:, pl.ds(base, _CHUNK)], bft_v)
            pltpu.sync_copy(eft_hbm.at[:, pl.ds(base, _CHUNK)], eft_v)
            pltpu.sync_copy(csm_hbm.at[pl.ds(base, _CHUNK)], csm_v)
            pltpu.sync_copy(al_hbm.at[pl.ds(base, _CHUNK)], al_v)
            cg.wait()

            @plsc.parallel_loop(0, _NGRP, 1)
            def group(g):
                r = g * 16 + lane
                acc_b0 = jnp.zeros((16,), jnp.float32)
                acc_b1 = jnp.zeros((16,), jnp.float32)
                acc_e0 = jnp.zeros((16,), jnp.float32)
                acc_e1 = jnp.zeros((16,), jnp.float32)
                for d in range(DB):
                    js = lax.rem(lane + d, jnp.full((16,), DB, jnp.int32))
                    jse = js + EOFF
                    pb = (plsc.load_gather(rows_v, [r, js])
                          * plsc.load_gather(bft_v, [js, r]))
                    pe = (plsc.load_gather(rows_v, [r, jse])
                          * plsc.load_gather(eft_v, [js, r]))
                    if d % 2 == 0:
                        acc_b0 += pb
                        acc_e0 += pe
                    else:
                        acc_b1 += pb
                        acc_e1 += pe
                acc_b = acc_b0 + acc_b1
                acc_e = acc_e0 + acc_e1
                baseline = jnp.minimum(jnp.exp(acc_b), 1e6)
                eff = 1.0 / (1.0 + jnp.exp(4.0 - acc_e))
                eff = jnp.clip(eff, 1e-6, 1.0 - 1e-6)
                sl = pl.ds(g * 16, 16)
                csm16 = csm_v[sl]
                al16 = al_v[sl]
                eff_v[sl] = eff
                base_v[sl] = baseline
                outc_v[sl] = csm16 * baseline * (1.0 - al16 * eff)

            pltpu.sync_copy(eff_v, eff_hbm.at[pl.ds(base, _CHUNK)])
            pltpu.sync_copy(base_v, base_hbm.at[pl.ds(base, _CHUNK)])
            pltpu.sync_copy(outc_v, outc_hbm.at[pl.ds(base, _CHUNK)])
            return carry

        lax.fori_loop(0, _NCHUNK, chunk, 0)

    return k(loc_ind, tbl, bft, eft, csm, alert)


def kernel(hosps, loc_ind, county_summer_mean, alert, baseline_features,
           eff_features, index, spatial_features,
           Wb1, bb1, Wb2, bb2, We1, be1, We2, be2):
    tbl = _mlp_table(spatial_features.T, Wb1, bb1, Wb2, bb2,
                     We1, be1, We2, be2)
    eff, base, outc = _sc_fused(loc_ind, tbl, baseline_features.T,
                                eff_features.T, county_summer_mean, alert)
    return jnp.stack([eff, base, outc], axis=1)
